# Initial kernel scaffold; baseline (speedup 1.0000x reference)
import jax
import jax.numpy as jnp
from jax.experimental import pallas as pl


def kernel(x, spa_emb_weight):
    raise NotImplementedError("write your pallas kernel here")


# speedup vs baseline: 1.0795x; 1.0795x over previous
import functools

import jax
import jax.numpy as jnp
from jax import lax
from jax.experimental import pallas as pl
from jax.experimental.pallas import tpu as pltpu
from jax.experimental.pallas import tpu_sc as plsc

D_MODEL = 32
CHUNK = 128


@functools.cache
def _build(B):
    info = plsc.get_sparse_core_info()
    NC, NS = info.num_cores, info.num_subcores
    NW = NC * NS
    b_per_w = B // NW
    n_chunks = b_per_w // CHUNK
    mesh = plsc.VectorSubcoreMesh(core_axis_name="c", subcore_axis_name="s")

    @functools.partial(
        pl.kernel,
        mesh=mesh,
        compiler_params=pltpu.CompilerParams(use_tc_tiling_on_sc=False),
        out_type=jax.ShapeDtypeStruct((B, D_MODEL), jnp.float32),
        scratch_types=[
            pltpu.VMEM((n_chunks, CHUNK), jnp.int32),
            pltpu.VMEM((CHUNK, D_MODEL), jnp.float32),
            pltpu.VMEM((CHUNK, D_MODEL), jnp.float32),
            pltpu.SemaphoreType.DMA,
            pltpu.SemaphoreType.DMA,
            pltpu.SemaphoreType.DMA,
            pltpu.SemaphoreType.DMA,
        ],
    )
    def emb_kernel(idx_hbm, table_hbm, out_hbm, idx_v, rows0, rows1,
                   g0, g1, w0, w1):
        wid = lax.axis_index("s") * NC + lax.axis_index("c")
        base = wid * b_per_w
        pltpu.sync_copy(idx_hbm.at[wid], idx_v)

        bufs = ((rows0, g0, w0), (rows1, g1, w1))

        pltpu.async_copy(table_hbm.at[idx_v.at[0]], rows0, g0)
        pltpu.async_copy(table_hbm.at[idx_v.at[1]], rows1, g1)

        def body(jj, carry):
            for b in range(2):
                rows, g, w = bufs[b]
                j = jj * 2 + b
                pltpu.make_async_copy(table_hbm.at[idx_v.at[0]], rows, g).wait()
                cp = pltpu.async_copy(
                    rows, out_hbm.at[pl.ds(base + j * CHUNK, CHUNK)], w)

                @pl.when(jj < n_chunks // 2 - 1)
                def _():
                    pltpu.make_async_copy(
                        rows, out_hbm.at[pl.ds(base, CHUNK)], w).wait()
                    pltpu.async_copy(table_hbm.at[idx_v.at[j + 2]], rows, g)

                @pl.when(jj == n_chunks // 2 - 1)
                def _():
                    pltpu.make_async_copy(
                        rows, out_hbm.at[pl.ds(base, CHUNK)], w).wait()

            return carry

        lax.fori_loop(0, n_chunks // 2, body, 0)

    return emb_kernel


def kernel(x, spa_emb_weight):
    Bm, H = x.shape
    B = Bm * H
    idx = x.reshape(-1).astype(jnp.int32)
    info = plsc.get_sparse_core_info()
    NW = info.num_cores * info.num_subcores
    idx3 = idx.reshape(NW, (B // NW) // CHUNK, CHUNK)
    out = _build(B)(idx3, spa_emb_weight)
    return out.reshape(Bm, H, D_MODEL)


# speedup vs baseline: 1.1138x; 1.0317x over previous
import functools

import jax
import jax.numpy as jnp
from jax import lax
from jax.experimental import pallas as pl
from jax.experimental.pallas import tpu as pltpu
from jax.experimental.pallas import tpu_sc as plsc

D_MODEL = 32
CHUNK = 128
CHUNKS_PER_BLK = 10
BLK = CHUNK * CHUNKS_PER_BLK


@functools.cache
def _build(B):
    info = plsc.get_sparse_core_info()
    NC, NS = info.num_cores, info.num_subcores
    NW = NC * NS
    b_per_w = B // NW
    n_chunks = b_per_w // CHUNK
    n_blocks = b_per_w // BLK
    assert n_blocks % 2 == 0 and n_blocks * BLK == b_per_w
    mesh = plsc.VectorSubcoreMesh(core_axis_name="c", subcore_axis_name="s")

    @functools.partial(
        pl.kernel,
        mesh=mesh,
        compiler_params=pltpu.CompilerParams(use_tc_tiling_on_sc=False),
        out_type=jax.ShapeDtypeStruct((B, D_MODEL), jnp.float32),
        scratch_types=[
            pltpu.VMEM((n_chunks, CHUNK), jnp.int32),
            pltpu.VMEM((BLK, D_MODEL), jnp.float32),
            pltpu.VMEM((BLK, D_MODEL), jnp.float32),
            pltpu.SemaphoreType.DMA,
            pltpu.SemaphoreType.DMA,
            pltpu.SemaphoreType.DMA,
            pltpu.SemaphoreType.DMA,
        ],
    )
    def emb_kernel(idx_hbm, table_hbm, out_hbm, idx_v, buf0, buf1,
                   g0, g1, w0, w1):
        wid = lax.axis_index("s") * NC + lax.axis_index("c")
        base = wid * b_per_w
        pltpu.sync_copy(idx_hbm.at[wid], idx_v)

        def fire(blk, buf, gsem):
            for c in range(CHUNKS_PER_BLK):
                pltpu.async_copy(
                    table_hbm.at[idx_v.at[blk * CHUNKS_PER_BLK + c]],
                    buf.at[pl.ds(c * CHUNK, CHUNK)], gsem)

        def drain_gathers(buf, gsem):
            pltpu.make_async_copy(
                out_hbm.at[pl.ds(0, BLK)], buf, gsem).wait()

        def wait_write(buf, wsem):
            pltpu.make_async_copy(
                buf, out_hbm.at[pl.ds(0, BLK)], wsem).wait()

        fire(0, buf0, g0)

        def body(gg, carry):
            for b, (buf, gsem, wsem) in enumerate(
                    ((buf0, g0, w0), (buf1, g1, w1))):
                blk = gg * 2 + b

                nxt = blk + 1
                obuf, ogsem, owsem = ((buf1, g1, w1), (buf0, g0, w0))[b]

                @pl.when(nxt < n_blocks)
                def _():
                    @pl.when(nxt >= 2)
                    def _():
                        wait_write(obuf, owsem)
                    fire(nxt, obuf, ogsem)

                drain_gathers(buf, gsem)
                pltpu.async_copy(
                    buf, out_hbm.at[pl.ds(base + blk * BLK, BLK)], wsem)
            return carry

        lax.fori_loop(0, n_blocks // 2, body, 0)
        wait_write(buf0, w0)
        wait_write(buf1, w1)

    return emb_kernel


def kernel(x, spa_emb_weight):
    Bm, H = x.shape
    B = Bm * H
    idx = x.reshape(-1).astype(jnp.int32)
    info = plsc.get_sparse_core_info()
    NW = info.num_cores * info.num_subcores
    idx3 = idx.reshape(NW, (B // NW) // CHUNK, CHUNK)
    out = _build(B)(idx3, spa_emb_weight)
    return out.reshape(Bm, H, D_MODEL)


# speedup vs baseline: 1.3833x; 1.2420x over previous
import functools

import jax
import jax.numpy as jnp
from jax import lax
from jax.experimental import pallas as pl
from jax.experimental.pallas import tpu as pltpu
from jax.experimental.pallas import tpu_sc as plsc

D_MODEL = 32
CHUNK = 128
NBUF = 4
L = 16


@functools.cache
def _build(batch, hist):
    info = plsc.get_sparse_core_info()
    NC, NS = info.num_cores, info.num_subcores
    NW = NC * NS
    n_bb = batch // CHUNK
    n_total = hist * n_bb
    t_per_w = n_total // NW
    assert t_per_w % NBUF == 0
    mesh = plsc.VectorSubcoreMesh(core_axis_name="c", subcore_axis_name="s")

    @functools.partial(
        pl.kernel,
        mesh=mesh,
        compiler_params=pltpu.CompilerParams(
            use_tc_tiling_on_sc=False, needs_layout_passes=False),
        out_type=jax.ShapeDtypeStruct((hist, D_MODEL, batch), jnp.float32),
        scratch_types=[
            pltpu.VMEM((t_per_w, CHUNK), jnp.int32),
            [pltpu.VMEM((CHUNK, D_MODEL), jnp.float32) for _ in range(NBUF)],
            [pltpu.VMEM((D_MODEL, CHUNK), jnp.float32) for _ in range(NBUF)],
            [pltpu.SemaphoreType.DMA for _ in range(NBUF)],
            [pltpu.SemaphoreType.DMA for _ in range(NBUF)],
        ],
    )
    def emb_kernel(idx_hbm, table_hbm, out_hbm, idx_v, rbufs, tbufs,
                   gsems, wsems):
        wid = lax.axis_index("s") * NC + lax.axis_index("c")
        base_t = wid * t_per_w
        pltpu.sync_copy(idx_hbm.at[wid], idx_v)

        iota = lax.iota(jnp.int32, L)

        def fire_gather(t, p):
            pltpu.async_copy(table_hbm.at[idx_v.at[t]], rbufs[p], gsems[p])

        def transpose(p):
            r, tb = rbufs[p], tbufs[p]
            for j in range(D_MODEL):
                col = jnp.full((L,), j, jnp.int32)
                for kb in range(CHUNK // L):
                    vals = plsc.load_gather(r, [iota + (kb * L), col])
                    tb[j, pl.ds(kb * L, L)] = vals

        def fire_write(t, p):
            c = base_t + t
            h = c // n_bb
            bb = c % n_bb
            pltpu.async_copy(
                tbufs[p], out_hbm.at[h, :, pl.ds(bb * CHUNK, CHUNK)],
                wsems[p])

        def wait_gather(p):
            pltpu.make_async_copy(
                out_hbm.at[0, :, pl.ds(0, CHUNK)], rbufs[p], gsems[p]).wait()

        def wait_write(p):
            pltpu.make_async_copy(
                tbufs[p], out_hbm.at[0, :, pl.ds(0, CHUNK)], wsems[p]).wait()

        for p in range(NBUF):
            fire_gather(p, p)

        def body(i, carry):
            for p in range(NBUF):
                t = i * NBUF + p
                wait_gather(p)

                @pl.when(t >= NBUF)
                def _():
                    wait_write(p)

                transpose(p)

                @pl.when(t + NBUF < t_per_w)
                def _():
                    fire_gather(t + NBUF, p)

                fire_write(t, p)
            return carry

        lax.fori_loop(0, t_per_w // NBUF, body, 0)
        for p in range(NBUF):
            wait_write(p)

    return emb_kernel


def kernel(x, spa_emb_weight):
    batch, hist = x.shape
    info = plsc.get_sparse_core_info()
    NW = info.num_cores * info.num_subcores
    n_idx_per_w = batch * hist // NW
    idx3 = x.T.astype(jnp.int32).reshape(NW, n_idx_per_w // CHUNK, CHUNK)
    outT = _build(batch, hist)(idx3, spa_emb_weight)
    return jnp.transpose(outT, (2, 0, 1))
